# Initial kernel scaffold; baseline (speedup 1.0000x reference)
import jax
import jax.numpy as jnp
from jax.experimental import pallas as pl


def kernel(x, edge_index, node_type_emb, gate_type_emb, enc_W, enc_b, convW, convb, bn_gamma, bn_beta, bn_mean, bn_var, att_W1, att_b1, att_W2, att_b2, gp_W1, gp_b1, gp_W2, gp_b2, ppa_W1, ppa_b1, ppa_W2, ppa_b2, val_W1, val_b1, val_W2, val_b2, conf_W1, conf_b1, conf_W2, conf_b2):
    raise NotImplementedError("write your pallas kernel here")


# speedup vs baseline: 5.1914x; 5.1914x over previous
import functools

import jax
import jax.numpy as jnp
from jax import lax
from jax.experimental import pallas as pl
from jax.experimental.pallas import tpu as pltpu
from jax.experimental.pallas import tpu_sc as plsc

N = 10000
H = 512
NP = 10240
NBLK = 4
RB = NP // NBLK
CCH = 4
RT = NP // 16
E = 160000
KB = 128
EPT = 10112
NB_E = EPT // KB
EPT32 = 5120

_MESH = dict(core_axis_name="c", subcore_axis_name="s")


def _sc_degree_counts(dst32, zeros_np):
    def body(dst_hbm, zeros_hbm, out_hbm, dst_all, counts):
        cid = lax.axis_index("c")
        sid = lax.axis_index("s")
        wid = sid * 2 + cid
        pltpu.sync_copy(zeros_hbm, counts)
        pltpu.sync_copy(dst_hbm.at[wid], dst_all)
        ones = jnp.full((16,), 1.0, jnp.float32)

        def step(j, carry):
            dvec = dst_all[pl.ds(j * 16, 16)]
            plsc.addupdate_scatter(counts, [dvec], ones)
            return carry

        lax.fori_loop(0, EPT32 // 16, step, 0)
        pltpu.sync_copy(counts, out_hbm.at[wid])

    fn = pl.kernel(
        body,
        out_type=jax.ShapeDtypeStruct((32, NP), jnp.float32),
        mesh=plsc.VectorSubcoreMesh(**_MESH),
        scratch_types=[
            pltpu.VMEM((EPT32,), jnp.int32),
            pltpu.VMEM((NP,), jnp.float32),
        ],
        compiler_params=pltpu.CompilerParams(needs_layout_passes=False),
    )
    return fn(dst32, zeros_np)


def _sc_segsum(hls, src16, dst16):
    def body(hls_hbm, src_hbm, dst_hbm, out_hbm,
             src_buf, dst_buf, rows, acc):
        cid = lax.axis_index("c")
        sid = lax.axis_index("s")
        for p in range(2):
            chunk = cid * 2 + p
            hls_c = hls_hbm.at[chunk]
            pltpu.sync_copy(hls_c.at[pl.ds(sid * RT, RT)],
                            acc.at[pl.ds(sid * RT, RT)])
            plsc.subcore_barrier()

            def step(b, carry):
                off = b * KB
                pltpu.sync_copy(src_hbm.at[sid, pl.ds(off, KB)], src_buf)
                pltpu.sync_copy(dst_hbm.at[sid, pl.ds(off, KB)], dst_buf)
                pltpu.sync_copy(hls_c.at[src_buf], rows)
                pltpu.sync_copy(rows, acc.at[dst_buf], add=True)
                return carry

            lax.fori_loop(0, NB_E, step, 0)
            plsc.subcore_barrier()
            pltpu.sync_copy(acc.at[pl.ds(sid * RT, RT)],
                            out_hbm.at[chunk, pl.ds(sid * RT, RT)])
            plsc.subcore_barrier()

    fn = pl.kernel(
        body,
        out_type=jax.ShapeDtypeStruct((4, NP, 128), jnp.float32),
        mesh=plsc.VectorSubcoreMesh(**_MESH),
        scratch_types=[
            pltpu.VMEM((KB,), jnp.int32),
            pltpu.VMEM((KB,), jnp.int32),
            pltpu.VMEM((KB, 128), jnp.float32),
            pltpu.VMEM_SHARED((NP, 128), jnp.float32),
        ],
        compiler_params=pltpu.CompilerParams(needs_layout_passes=False),
    )
    return fn(hls, src16, dst16)


def _first_max_onehot_cols(sub, k):
    m = jnp.max(sub, axis=1, keepdims=True)
    cols = []
    taken = None
    for j in range(k):
        ej = sub[:, j:j + 1] == m
        if taken is None:
            fj, taken = ej, ej
        else:
            fj = jnp.logical_and(ej, jnp.logical_not(taken))
            taken = jnp.logical_or(taken, ej)
        cols.append(fj.astype(jnp.float32))
    return cols


def _encode_body(x_ref, nt_ref, gt_ref, wx_ref, wnt_ref, wgt_ref, b_ref,
                 out_ref):
    x = x_ref[...]
    t_nt = jnp.dot(nt_ref[...], wnt_ref[...],
                   preferred_element_type=jnp.float32)
    t_gt = jnp.dot(gt_ref[...], wgt_ref[...],
                   preferred_element_type=jnp.float32)
    acc = jnp.dot(x, wx_ref[...], preferred_element_type=jnp.float32)
    acc = acc + b_ref[...]
    for j, fj in enumerate(_first_max_onehot_cols(x[:, 0:4], 4)):
        acc = acc + fj * t_nt[j:j + 1, :]
    for j, fj in enumerate(_first_max_onehot_cols(x[:, 4:12], 8)):
        acc = acc + fj * t_gt[j:j + 1, :]
    out_ref[...] = jnp.maximum(acc, 0.0)


def _encode(xp, node_type_emb, gate_type_emb, enc_W, enc_b):
    wx, wnt, wgt = enc_W[:24], enc_W[24:40], enc_W[40:56]
    return pl.pallas_call(
        _encode_body,
        grid=(NBLK,),
        in_specs=[
            pl.BlockSpec((RB, 24), lambda r: (r, 0)),
            pl.BlockSpec((4, 16), lambda r: (0, 0)),
            pl.BlockSpec((9, 16), lambda r: (0, 0)),
            pl.BlockSpec((24, H), lambda r: (0, 0)),
            pl.BlockSpec((16, H), lambda r: (0, 0)),
            pl.BlockSpec((16, H), lambda r: (0, 0)),
            pl.BlockSpec((1, H), lambda r: (0, 0)),
        ],
        out_specs=pl.BlockSpec((RB, H), lambda r: (r, 0)),
        out_shape=jax.ShapeDtypeStruct((NP, H), jnp.float32),
    )(xp, node_type_emb, gate_type_emb, wx, wnt, wgt, enc_b[None])


def _dinv_body(cnt_ref, out_ref):
    r = pl.program_id(0)
    deg = 1.0 + jnp.sum(cnt_ref[...], axis=0)
    dv = lax.rsqrt(deg)[:, None]
    rowid = lax.broadcasted_iota(jnp.int32, (RB, 1), 0) + r * RB
    out_ref[...] = jnp.where(rowid < N, dv, 0.0)


def _dinv(counts32):
    return pl.pallas_call(
        _dinv_body,
        grid=(NBLK,),
        in_specs=[pl.BlockSpec((32, RB), lambda r: (0, r))],
        out_specs=pl.BlockSpec((RB, 1), lambda r: (r, 0)),
        out_shape=jax.ShapeDtypeStruct((NP, 1), jnp.float32),
    )(counts32)


def _mm0_body(h_ref, dv_ref, w_ref, out_ref):
    out_ref[0] = jnp.dot(h_ref[...], w_ref[...],
                         preferred_element_type=jnp.float32) * dv_ref[...]


def _layer0_hls(h0, dinv, w):
    return pl.pallas_call(
        _mm0_body,
        grid=(NBLK, CCH),
        in_specs=[
            pl.BlockSpec((RB, H), lambda r, c: (r, 0)),
            pl.BlockSpec((RB, 1), lambda r, c: (r, 0)),
            pl.BlockSpec((H, 128), lambda r, c: (0, c)),
        ],
        out_specs=pl.BlockSpec((1, RB, 128), lambda r, c: (c, r, 0)),
        out_shape=jax.ShapeDtypeStruct((CCH, NP, 128), jnp.float32),
    )(h0, dinv, w)


def _post_bn_relu(agg, dv, cb, gam, bet, mu, var):
    hcat = jnp.concatenate([agg[0], agg[1], agg[2], agg[3]], axis=-1)
    v = hcat * dv + cb
    s = gam * lax.rsqrt(var + 1e-5)
    return jnp.maximum((v - mu) * s + bet, 0.0)


def _mmi_body(agg_ref, dv_ref, cb_ref, g_ref, b_ref, m_ref, v_ref, w_ref,
              out_ref):
    h = _post_bn_relu(agg_ref[...], dv_ref[...], cb_ref[...], g_ref[...],
                      b_ref[...], m_ref[...], v_ref[...])
    out_ref[0] = jnp.dot(h, w_ref[...],
                         preferred_element_type=jnp.float32) * dv_ref[...]


def _layeri_hls(agg, dinv, cb, gam, bet, mu, var, w):
    vec = lambda a: a[None]
    return pl.pallas_call(
        _mmi_body,
        grid=(NBLK, CCH),
        in_specs=[
            pl.BlockSpec((CCH, RB, 128), lambda r, c: (0, r, 0)),
            pl.BlockSpec((RB, 1), lambda r, c: (r, 0)),
            pl.BlockSpec((1, H), lambda r, c: (0, 0)),
            pl.BlockSpec((1, H), lambda r, c: (0, 0)),
            pl.BlockSpec((1, H), lambda r, c: (0, 0)),
            pl.BlockSpec((1, H), lambda r, c: (0, 0)),
            pl.BlockSpec((1, H), lambda r, c: (0, 0)),
            pl.BlockSpec((H, 128), lambda r, c: (0, c)),
        ],
        out_specs=pl.BlockSpec((1, RB, 128), lambda r, c: (c, r, 0)),
        out_shape=jax.ShapeDtypeStruct((CCH, NP, 128), jnp.float32),
    )(agg, dinv, vec(cb), vec(gam), vec(bet), vec(mu), vec(var), w)


def _pool_head_body(agg_ref, dv_ref, cb_ref, g_ref, b_ref, m_ref, v_ref,
                    aw1_ref, ab1_ref, aw2_ref, ab2_ref,
                    gw1_ref, gb1_ref, gw2_ref, gb2_ref,
                    pw1_ref, pb1_ref, pw2_ref, pb2_ref,
                    vw1_ref, vb1_ref, vw2_ref, vb2_ref,
                    cw1_ref, cb1_ref, cw2_ref, cb2_ref,
                    g_out, ppa_out, val_out, conf_out,
                    ssum, asum):
    r = pl.program_id(0)
    h = _post_bn_relu(agg_ref[...], dv_ref[...], cb_ref[...], g_ref[...],
                      b_ref[...], m_ref[...], v_ref[...])
    a1 = jnp.maximum(jnp.dot(h, aw1_ref[...],
                             preferred_element_type=jnp.float32)
                     + ab1_ref[...], 0.0)
    att = jax.nn.sigmoid(jnp.dot(a1, aw2_ref[...],
                                 preferred_element_type=jnp.float32)
                         + ab2_ref[...])
    rowid = lax.broadcasted_iota(jnp.int32, (RB, 1), 0) + r * RB
    att = jnp.where(rowid < N, att, 0.0)
    part = jnp.sum(h * att, axis=0, keepdims=True)
    apart = jnp.sum(att)

    @pl.when(r == 0)
    def _():
        ssum[...] = part
        asum[0] = apart

    @pl.when(r > 0)
    def _():
        ssum[...] = ssum[...] + part
        asum[0] = asum[0] + apart

    @pl.when(r == NBLK - 1)
    def _():
        pooled = ssum[...] / asum[0]
        g1 = jnp.maximum(jnp.dot(pooled, gw1_ref[...],
                                 preferred_element_type=jnp.float32)
                         + gb1_ref[...], 0.0)
        g = jnp.dot(g1, gw2_ref[...],
                    preferred_element_type=jnp.float32) + gb2_ref[...]
        g_out[...] = g

        def head(w1, b1, w2, b2):
            t = jnp.maximum(jnp.dot(g, w1[...],
                                    preferred_element_type=jnp.float32)
                            + b1[...], 0.0)
            return jnp.dot(t, w2[...],
                           preferred_element_type=jnp.float32) + b2[...]

        ppa_out[...] = jax.nn.sigmoid(head(pw1_ref, pb1_ref, pw2_ref,
                                           pb2_ref))
        val_out[...] = head(vw1_ref, vb1_ref, vw2_ref, vb2_ref)
        conf_out[...] = jax.nn.sigmoid(head(cw1_ref, cb1_ref, cw2_ref,
                                            cb2_ref))


def _pool_heads(agg, dinv, cb, gam, bet, mu, var,
                att_W1, att_b1, att_W2, att_b2,
                gp_W1, gp_b1, gp_W2, gp_b2,
                ppa_W1, ppa_b1, ppa_W2, ppa_b2,
                val_W1, val_b1, val_W2, val_b2,
                conf_W1, conf_b1, conf_W2, conf_b2):
    vec = lambda a: a[None]
    full = lambda a: pl.BlockSpec(a.shape, lambda r: tuple(0 for _ in a.shape))
    ins = [agg, dinv, vec(cb), vec(gam), vec(bet), vec(mu), vec(var),
           att_W1, vec(att_b1), att_W2, vec(att_b2),
           gp_W1, vec(gp_b1), gp_W2, vec(gp_b2),
           ppa_W1, vec(ppa_b1), ppa_W2, vec(ppa_b2),
           val_W1, vec(val_b1), val_W2, vec(val_b2),
           conf_W1, vec(conf_b1), conf_W2, vec(conf_b2)]
    in_specs = ([pl.BlockSpec((CCH, RB, 128), lambda r: (0, r, 0)),
                 pl.BlockSpec((RB, 1), lambda r: (r, 0))]
                + [full(a) for a in ins[2:]])
    return pl.pallas_call(
        _pool_head_body,
        grid=(NBLK,),
        in_specs=in_specs,
        out_specs=[
            pl.BlockSpec((1, H), lambda r: (0, 0)),
            pl.BlockSpec((1, 3), lambda r: (0, 0)),
            pl.BlockSpec((1, 1), lambda r: (0, 0)),
            pl.BlockSpec((1, 1), lambda r: (0, 0)),
        ],
        out_shape=[
            jax.ShapeDtypeStruct((1, H), jnp.float32),
            jax.ShapeDtypeStruct((1, 3), jnp.float32),
            jax.ShapeDtypeStruct((1, 1), jnp.float32),
            jax.ShapeDtypeStruct((1, 1), jnp.float32),
        ],
        scratch_shapes=[
            pltpu.VMEM((1, H), jnp.float32),
            pltpu.SMEM((1,), jnp.float32),
        ],
    )(*ins)


def kernel(x, edge_index, node_type_emb, gate_type_emb, enc_W, enc_b,
           convW, convb, bn_gamma, bn_beta, bn_mean, bn_var,
           att_W1, att_b1, att_W2, att_b2, gp_W1, gp_b1, gp_W2, gp_b2,
           ppa_W1, ppa_b1, ppa_W2, ppa_b2, val_W1, val_b1, val_W2, val_b2,
           conf_W1, conf_b1, conf_W2, conf_b2):
    src = edge_index[0]
    dst = edge_index[1]
    src16 = jnp.pad(src, (0, 16 * EPT - E), constant_values=N).reshape(16, EPT)
    dst16 = jnp.pad(dst, (0, 16 * EPT - E), constant_values=N).reshape(16, EPT)
    dst32 = jnp.pad(dst, (0, 32 * EPT32 - E),
                    constant_values=N).reshape(32, EPT32)
    xp = jnp.pad(x, ((0, NP - N), (0, 0)))
    zeros_np = jnp.zeros((NP,), jnp.float32)

    h0 = _encode(xp, node_type_emb, gate_type_emb, enc_W, enc_b)
    counts32 = _sc_degree_counts(dst32, zeros_np)
    dinv = _dinv(counts32)

    agg = None
    for i in range(4):
        if i == 0:
            hls = _layer0_hls(h0, dinv, convW[0])
        else:
            hls = _layeri_hls(agg, dinv, convb[i - 1], bn_gamma[i - 1],
                              bn_beta[i - 1], bn_mean[i - 1], bn_var[i - 1],
                              convW[i])
        agg = _sc_segsum(hls, src16, dst16)

    g, ppa, val, conf = _pool_heads(
        agg, dinv, convb[3], bn_gamma[3], bn_beta[3], bn_mean[3], bn_var[3],
        att_W1, att_b1, att_W2, att_b2, gp_W1, gp_b1, gp_W2, gp_b2,
        ppa_W1, ppa_b1, ppa_W2, ppa_b2, val_W1, val_b1, val_W2, val_b2,
        conf_W1, conf_b1, conf_W2, conf_b2)
    return (g, ppa, val, conf)
